# Initial kernel scaffold; baseline (speedup 1.0000x reference)
import jax
import jax.numpy as jnp
from jax.experimental import pallas as pl


def kernel(x, enc_w1, enc_b1, bn1_g, bn1_b, enc_w2, enc_b2, bn2_g, bn2_b, enc_w3, enc_b3, bn3_g, bn3_b, codebook, dec_wt, dbn1_g, dbn1_b, dec_w2, dec_b2, dbn2_g, dbn2_b, dec_w3, dec_b3):
    raise NotImplementedError("write your pallas kernel here")


# speedup vs baseline: 1.6860x; 1.6860x over previous
import functools

import jax
import jax.numpy as jnp
from jax.experimental import pallas as pl

EPS = 1e-5
COMMIT_W = 2.0
F32 = jnp.float32


def _silu(v):
    return v * jax.nn.sigmoid(v)


HIGHEST = jax.lax.Precision.HIGHEST
BF16 = jnp.bfloat16


def _split3(x):
    hi = x.astype(BF16)
    r = x - hi.astype(F32)
    mid = r.astype(BF16)
    lo = (r - mid.astype(F32)).astype(BF16)
    return hi, mid, lo


_PAIRS = ((0, 0), (0, 1), (1, 0), (0, 2), (2, 0), (1, 1))


def _phase_shift(w):
    p = w % 5
    return p, (w - p) // 5


def _k1(x_ref, w1_ref, b1_ref, h_ref, st_ref):
    xp = x_ref[0].astype(BF16).astype(F32)
    U = xp.shape[1]
    xpp = jnp.pad(xp, ((0, 0), (1, 1)))
    ssum = jnp.zeros((16, 1), F32)
    ssq = jnp.zeros((16, 1), F32)
    for r in range(5):
        acc = jnp.zeros((16, U), F32)
        for j in range(5):
            p, s = _phase_shift(r + j - 2)
            wj = w1_ref[:, j:j + 1].astype(BF16).astype(F32)
            acc = acc + wj * xpp[p:p + 1, 1 + s:1 + s + U]
        h = _silu(acc + b1_ref[:, 0:1])
        h_ref[0, r] = h
        ssum = ssum + jnp.sum(h, axis=1, keepdims=True)
        ssq = ssq + jnp.sum(h * h, axis=1, keepdims=True)
    st_ref[0, :, 0:1] = ssum
    st_ref[0, :, 1:2] = ssq


def _k2(h1_ref, sc_ref, w2_ref, b2_ref, h_ref, st_ref):
    a = sc_ref[:, 0:1][None]
    c = sc_ref[:, 1:2][None]
    hn = h1_ref[0] * a + c
    U = hn.shape[2]
    hpp = jnp.pad(hn, ((0, 0), (0, 0), (1, 1)))
    ssum = jnp.zeros((32, 1), F32)
    ssq = jnp.zeros((32, 1), F32)
    for r in range(5):
        acc = jnp.zeros((32, U), F32)
        for m in range(5):
            p, s = _phase_shift(r + m - 2)
            acc = acc + jax.lax.dot(
                w2_ref[m], hpp[p, :, 1 + s:1 + s + U],
                preferred_element_type=F32)
        h = _silu(acc + b2_ref[:, 0:1])
        h_ref[0, r] = h
        ssum = ssum + jnp.sum(h, axis=1, keepdims=True)
        ssq = ssq + jnp.sum(h * h, axis=1, keepdims=True)
    st_ref[0, :, 0:1] = ssum
    st_ref[0, :, 1:2] = ssq


def _k3(h2_ref, sc_ref, w3_ref, b3_ref, z_ref, st_ref):
    a = sc_ref[:, 0:1][None]
    c = sc_ref[:, 1:2][None]
    hn = h2_ref[0] * a + c
    U = hn.shape[2]
    hpp = jnp.pad(hn, ((0, 0), (0, 0), (3, 3)))
    ssum = jnp.zeros((64, 1), F32)
    ssq = jnp.zeros((64, 1), F32)
    cw = U // 4
    for t in range(4):
        acc = jnp.zeros((64, cw), F32)
        for j in range(25):
            p, s = _phase_shift(j - 12)
            o = 3 + s + t * cw
            acc = acc + jax.lax.dot(
                w3_ref[j], hpp[p, :, o:o + cw],
                preferred_element_type=F32)
        z = jnp.tanh(acc + b3_ref[:, 0:1])
        z_ref[0, :, t * cw:(t + 1) * cw] = z
        ssum = ssum + jnp.sum(z, axis=1, keepdims=True)
        ssq = ssq + jnp.sum(z * z, axis=1, keepdims=True)
    st_ref[0, :, 0:1] = ssum
    st_ref[0, :, 1:2] = ssq


def _k4(z_ref, sc_ref, cb_ref, cbt_ref, cn_ref, idx_ref, q_ref, st_ref,
        *, n_chunk):
    a = sc_ref[:, 0:1]
    c = sc_ref[:, 1:2]
    zn = z_ref[0] * a + c
    U = zn.shape[1]
    cw = U // n_chunk
    dsum = jnp.zeros((1, 1), F32)
    zsum = jnp.zeros((1, 1), F32)
    for t in range(n_chunk):
        zc = zn[:, t * cw:(t + 1) * cw]
        zt = jnp.transpose(zc)
        dp = -2.0 * jax.lax.dot(zt, cbt_ref[...],
                                preferred_element_type=F32)
        dp = dp + cn_ref[0:1, :]
        idx = jnp.argmin(dp, axis=1)[:, None].astype(jnp.int32)
        idx_ref[0, t * cw:(t + 1) * cw, :] = idx
        dmin = jnp.min(dp, axis=1, keepdims=True)
        dsum = dsum + jnp.sum(dmin, axis=0, keepdims=True)
        zsum = zsum + jnp.sum(
            jnp.sum(zc * zc, axis=0, keepdims=True), axis=1, keepdims=True)
        oh = (jax.lax.broadcasted_iota(jnp.int32, (cw, 512), 1)
              == idx).astype(BF16)
        qc = jnp.zeros((cw, 64), F32)
        for i in range(3):
            qc = qc + jax.lax.dot(oh, cb_ref[i],
                                  preferred_element_type=F32)
        q_ref[0, :, t * cw:(t + 1) * cw] = jnp.transpose(qc)
    st_ref[0, 0:1, 0:1] = dsum
    st_ref[0, 0:1, 1:2] = zsum


def _k5(q_ref, wt_ref, h_ref, st_ref):
    vid = pl.program_id(1)
    q = q_ref[0]
    U = q.shape[1]
    qp = jnp.pad(q, ((0, 0), (3, 3)))
    lanes = jax.lax.broadcasted_iota(jnp.int32, (1, U), 1)
    lastmask = (lanes != U - 1).astype(F32)

    @pl.when(vid == 0)
    def _init():
        st_ref[0] = jnp.zeros_like(st_ref[0])

    for v in range(5):
        @pl.when(vid == v)
        def _phase(v=v):
            acc = jnp.zeros((64, U), F32)
            for anum in range(5):
                j = (v + 2) % 5 + 5 * anum
                d = (j - 12 - v) // 5
                acc = acc + jax.lax.dot(
                    wt_ref[v, anum], qp[:, 3 - d:3 - d + U],
                    preferred_element_type=F32)
            h = _silu(acc)
            if v >= 1:
                h = h * lastmask
            h_ref[0, 0] = h
            st_ref[0, :, 0:1] += jnp.sum(h, axis=1, keepdims=True)
            st_ref[0, :, 1:2] += jnp.sum(h * h, axis=1, keepdims=True)


def _k6(h_ref, w2_ref, bias_ref, dlt_ref, o_ref, st_ref):
    vid = pl.program_id(1)
    U = h_ref.shape[3]

    @pl.when(vid == 0)
    def _init():
        st_ref[0] = jnp.zeros_like(st_ref[0])

    for v in range(5):
        @pl.when(vid == v)
        def _phase(v=v):
            acc = jnp.zeros((64, U), F32) + bias_ref[:, 0:1]
            for m in range(5):
                p, s = _phase_shift(v + m - 2)
                hp = jnp.pad(h_ref[0, p], ((0, 0), (1, 1)))
                acc = acc + jax.lax.dot(
                    w2_ref[m], hp[:, 1 + s:1 + s + U],
                    preferred_element_type=F32)
            c0 = _silu(acc[:, 0:1] - dlt_ref[v, :, 0:1])
            cu2 = _silu(acc[:, U - 2:U - 1] - dlt_ref[v, :, 1:2])
            if v == 0:
                cu1 = _silu(acc[:, U - 1:U] - dlt_ref[v, :, 2:3])
            else:
                cu1 = jnp.zeros((64, 1), F32)
            h = _silu(acc)
            ssum = (jnp.sum(h, axis=1, keepdims=True)
                    - h[:, 0:1] - h[:, U - 2:U - 1] - h[:, U - 1:U]
                    + c0 + cu2 + cu1)
            ssq = (jnp.sum(h * h, axis=1, keepdims=True)
                   - h[:, 0:1] ** 2 - h[:, U - 2:U - 1] ** 2
                   - h[:, U - 1:U] ** 2 + c0 * c0 + cu2 * cu2 + cu1 * cu1)
            o_ref[0, 0] = h
            o_ref[0, 0, :, 0:1] = c0
            o_ref[0, 0, :, U - 2:U - 1] = cu2
            o_ref[0, 0, :, U - 1:U] = cu1
            st_ref[0, :, 0:1] += ssum
            st_ref[0, :, 1:2] += ssq


def _k7(h_ref, sc_ref, w3_ref, b3_ref, o_ref):
    a = sc_ref[:, 0:1]
    c = sc_ref[:, 1:2]
    U = h_ref.shape[3]
    lanes = jax.lax.broadcasted_iota(jnp.int32, (1, U), 1)
    lastmask = (lanes != U - 1).astype(F32)
    for v in range(5):
        hn = h_ref[0, v] * a + c
        row = jax.lax.dot(w3_ref[...], hn,
                          preferred_element_type=F32) + b3_ref[0, 0]
        if v >= 1:
            row = row * lastmask
        o_ref[0, v:v + 1, :] = row


def _scale_shift(st, g, b, n):
    s = jnp.sum(st[:, :, 0], axis=0)
    ss = jnp.sum(st[:, :, 1], axis=0)
    m = s / n
    v = ss / n - m * m
    a = g / jnp.sqrt(v + EPS)
    return jnp.stack([a, b - m * a], axis=1)


def kernel(x, enc_w1, enc_b1, bn1_g, bn1_b, enc_w2, enc_b2, bn2_g, bn2_b,
           enc_w3, enc_b3, bn3_g, bn3_b, codebook, dec_wt, dbn1_g, dbn1_b,
           dec_w2, dec_b2, dbn2_g, dbn2_b, dec_w3, dec_b3):
    B, _, T = x.shape
    U = T // 5
    K, D = codebook.shape
    Td = 5 * U - 4

    x_p = jnp.transpose(x.reshape(B, U, 5), (0, 2, 1))
    w1 = enc_w1.reshape(16, 5)
    b1 = enc_b1[:, None]
    w2 = jnp.transpose(enc_w2, (2, 0, 1))
    b2 = enc_b2[:, None]
    w3 = jnp.transpose(enc_w3, (2, 0, 1))
    b3 = enc_b3[:, None]
    j_idx = (jnp.arange(5)[:, None] + 2) % 5 + 5 * jnp.arange(5)[None, :]
    wt_p = jnp.transpose(dec_wt[:, :, j_idx], (2, 3, 1, 0))
    w2d = jnp.transpose(dec_w2, (2, 0, 1))
    b2d = dec_b2[:, None]
    w3d = dec_w3.reshape(1, 64)
    b3d = dec_b3[:, None]
    cbt = codebook.T
    cb3 = jnp.stack(_split3(codebook))
    cnorm = jnp.sum(codebook * codebook, axis=1)[None, :]

    fullspec = lambda shape: pl.BlockSpec(shape, lambda b, *_: (0,) * len(shape))
    bspec = lambda shape: pl.BlockSpec((1,) + shape,
                                       lambda b, *_: (b,) + (0,) * len(shape))

    h1, st1 = pl.pallas_call(
        _k1,
        grid=(B,),
        in_specs=[bspec((5, U)), fullspec((16, 5)), fullspec((16, 1))],
        out_specs=[bspec((5, 16, U)), bspec((16, 2))],
        out_shape=[jax.ShapeDtypeStruct((B, 5, 16, U), F32),
                   jax.ShapeDtypeStruct((B, 16, 2), F32)],
    )(x_p, w1, b1)
    sc1 = _scale_shift(st1, bn1_g, bn1_b, float(B * T))

    h2, st2 = pl.pallas_call(
        _k2,
        grid=(B,),
        in_specs=[bspec((5, 16, U)), fullspec((16, 2)),
                  fullspec((5, 32, 16)), fullspec((32, 1))],
        out_specs=[bspec((5, 32, U)), bspec((32, 2))],
        out_shape=[jax.ShapeDtypeStruct((B, 5, 32, U), F32),
                   jax.ShapeDtypeStruct((B, 32, 2), F32)],
    )(h1, sc1, w2, b2)
    sc2 = _scale_shift(st2, bn2_g, bn2_b, float(B * T))

    z, st3 = pl.pallas_call(
        _k3,
        grid=(B,),
        in_specs=[bspec((5, 32, U)), fullspec((32, 2)),
                  fullspec((25, 64, 32)), fullspec((64, 1))],
        out_specs=[bspec((64, U)), bspec((64, 2))],
        out_shape=[jax.ShapeDtypeStruct((B, 64, U), F32),
                   jax.ShapeDtypeStruct((B, 64, 2), F32)],
    )(h2, sc2, w3, b3)
    sc3 = _scale_shift(st3, bn3_g, bn3_b, float(B * U))

    idx3, q, st4 = pl.pallas_call(
        functools.partial(_k4, n_chunk=4),
        grid=(B,),
        in_specs=[bspec((64, U)), fullspec((64, 2)), fullspec((3, K, D)),
                  fullspec((D, K)), fullspec((1, K))],
        out_specs=[bspec((U, 1)), bspec((64, U)), bspec((1, 2))],
        out_shape=[jax.ShapeDtypeStruct((B, U, 1), jnp.int32),
                   jax.ShapeDtypeStruct((B, 64, U), F32),
                   jax.ShapeDtypeStruct((B, 1, 2), F32)],
    )(z, sc3, cb3, cbt, cnorm)
    commit = COMMIT_W * (jnp.sum(st4[:, 0, 0]) + jnp.sum(st4[:, 0, 1])) \
        / float(B * U * D)

    hd1, st5 = pl.pallas_call(
        _k5,
        grid=(B, 5),
        in_specs=[pl.BlockSpec((1, 64, U), lambda b, v: (b, 0, 0)),
                  pl.BlockSpec((5, 5, 64, 64), lambda b, v: (0, 0, 0, 0))],
        out_specs=[pl.BlockSpec((1, 1, 64, U), lambda b, v: (b, v, 0, 0)),
                   pl.BlockSpec((1, 64, 2), lambda b, v: (b, 0, 0))],
        out_shape=[jax.ShapeDtypeStruct((B, 5, 64, U), F32),
                   jax.ShapeDtypeStruct((B, 64, 2), F32)],
    )(q, wt_p)
    sc5 = _scale_shift(st5, dbn1_g, dbn1_b, float(B * Td))

    a5, c5 = sc5[:, 0], sc5[:, 1]
    w6 = w2d * a5[None, None, :]
    corr = jnp.einsum('moi,i->mo', w2d, c5)
    bias6 = jnp.sum(corr, axis=0)[:, None] + b2d
    drows = []
    for v in range(5):
        d0 = jnp.zeros((64,), F32)
        du2 = jnp.zeros((64,), F32)
        du1 = jnp.zeros((64,), F32)
        for m in range(5):
            p, s = _phase_shift(v + m - 2)
            if s == -1:
                d0 = d0 + corr[m]
            if s == 1:
                du1 = du1 + corr[m]
            if s == 1 and p >= 1:
                du2 = du2 + corr[m]
            if s == 0 and p >= 1:
                du1 = du1 + corr[m]
        drows.append(jnp.stack([d0, du2, du1], axis=1))
    dlt = jnp.stack(drows)
    hd2, st6 = pl.pallas_call(
        _k6,
        grid=(B, 5),
        in_specs=[pl.BlockSpec((1, 5, 64, U), lambda b, v: (b, 0, 0, 0)),
                  pl.BlockSpec((5, 64, 64), lambda b, v: (0, 0, 0)),
                  pl.BlockSpec((64, 1), lambda b, v: (0, 0)),
                  pl.BlockSpec((5, 64, 3), lambda b, v: (0, 0, 0))],
        out_specs=[pl.BlockSpec((1, 1, 64, U), lambda b, v: (b, v, 0, 0)),
                   pl.BlockSpec((1, 64, 2), lambda b, v: (b, 0, 0))],
        out_shape=[jax.ShapeDtypeStruct((B, 5, 64, U), F32),
                   jax.ShapeDtypeStruct((B, 64, 2), F32)],
    )(hd1, w6, bias6, dlt)
    sc6 = _scale_shift(st6, dbn2_g, dbn2_b, float(B * Td))

    rp = pl.pallas_call(
        _k7,
        grid=(B,),
        in_specs=[bspec((5, 64, U)), fullspec((64, 2)),
                  fullspec((1, 64)), fullspec((1, 1))],
        out_specs=bspec((5, U)),
        out_shape=jax.ShapeDtypeStruct((B, 5, U), F32),
    )(hd2, sc6, w3d, b3d)

    recon = jnp.transpose(rp, (0, 2, 1)).reshape(B, 1, T)
    indices = idx3.reshape(B, U)
    return (recon, indices, commit)
